# baseline (speedup 1.0000x reference)
import dataclasses
import functools

import jax
import jax.numpy as jnp
from jax import lax
from jax.experimental import pallas as pl
from jax.experimental.pallas import tpu as pltpu
from jax.experimental.pallas import tpu_sc as plsc

N = 10000
E = 160000
D = 256
H = 128

NC = 2
NS = 16
E_PAD = 163840
B = 128

_mesh = plsc.VectorSubcoreMesh(
    core_axis_name="c", subcore_axis_name="s", num_cores=NC, num_subcores=NS
)

_sc_params = pltpu.CompilerParams()
if "needs_layout_passes" in pltpu.CompilerParams.__dataclass_fields__:
    _sc_params = dataclasses.replace(_sc_params, needs_layout_passes=False)


EPT1 = E_PAD // (NC * NS)


@functools.partial(
    pl.kernel,
    out_type=jax.ShapeDtypeStruct((NC * NS * N,), jnp.float32),
    mesh=_mesh,
    scratch_types=[
        pltpu.VMEM((N,), jnp.float32),
        pltpu.VMEM((EPT1,), jnp.int32),
        pltpu.VMEM((EPT1,), jnp.float32),
    ],
    compiler_params=_sc_params,
)
def _deg_kernel(dst_hbm, ew_hbm, out_hbm, part_v, dst_v, ew_v):
    wid = lax.axis_index("s") * NC + lax.axis_index("c")
    base = wid * EPT1

    @pl.loop(0, N, step=16)
    def _zero(i):
        part_v[pl.ds(i, 16)] = jnp.zeros((16,), jnp.float32)

    pltpu.sync_copy(dst_hbm.at[pl.ds(base, EPT1)], dst_v)
    pltpu.sync_copy(ew_hbm.at[pl.ds(base, EPT1)], ew_v)

    @pl.loop(0, EPT1, step=16)
    def _acc(k):
        plsc.addupdate_scatter(part_v, [dst_v[pl.ds(k, 16)]], ew_v[pl.ds(k, 16)])

    pltpu.sync_copy(part_v, out_hbm.at[pl.ds(wid * N, N)])


R2 = 1000


def _mm_body(node_ref, w_ref, degp_ref, h2a_ref, h2b_ref, dinv_ref):
    deg = jnp.sum(degp_ref[...], axis=1) + 1.0
    dinv = jnp.where(deg > 0, lax.rsqrt(deg), 0.0)
    h = jnp.dot(node_ref[...], w_ref[...], preferred_element_type=jnp.float32)
    h2 = h * dinv[:, None]
    h2a_ref[...] = h2[:, :H]
    h2b_ref[...] = h2[:, H:]
    dinv_ref[...] = dinv[:, None]


def _matmul_scale(node, w, deg_parts):
    return pl.pallas_call(
        _mm_body,
        grid=(N // R2,),
        in_specs=[
            pl.BlockSpec((R2, D), lambda i: (i, 0)),
            pl.BlockSpec((D, D), lambda i: (0, 0)),
            pl.BlockSpec((R2, NC * NS), lambda i: (i, 0)),
        ],
        out_specs=[
            pl.BlockSpec((R2, H), lambda i: (i, 0)),
            pl.BlockSpec((R2, H), lambda i: (i, 0)),
            pl.BlockSpec((R2, 1), lambda i: (i, 0)),
        ],
        out_shape=[
            jax.ShapeDtypeStruct((N, H), jnp.float32),
            jax.ShapeDtypeStruct((N, H), jnp.float32),
            jax.ShapeDtypeStruct((N, 1), jnp.float32),
        ],
    )(node, w, deg_parts)


EPT3 = E_PAD // NS
NSTG = 2
NCH = EPT3 // (B * NSTG)
NBUF = 2
NSPL = 4
ESTG = NCH * B
RPT = 624
TAIL0 = NS * RPT
TAIL = N - TAIL0


@functools.partial(
    pl.kernel,
    out_type=(
        jax.ShapeDtypeStruct((N, H), jnp.float32),
        jax.ShapeDtypeStruct((N, H), jnp.float32),
    ),
    mesh=_mesh,
    scratch_types=[
        pltpu.VMEM_SHARED((N, H), jnp.float32),
        pltpu.VMEM((ESTG,), jnp.int32),
        pltpu.VMEM((NCH, B), jnp.int32),
        pltpu.VMEM((ESTG,), jnp.float32),
        [pltpu.VMEM((B, H), jnp.float32) for _ in range(NBUF)],
        pltpu.SemaphoreType.DMA((NBUF * NSPL,)),
        pltpu.SemaphoreType.DMA((NBUF,)),
        pltpu.SemaphoreType.DMA,
    ],
    compiler_params=_sc_params,
)
def _agg_kernel(h2a_hbm, h2b_hbm, src_hbm, dst_hbm, ew_hbm,
                outa_hbm, outb_hbm, acc_sh, src_v, dst_v, ew_v, rows,
                gsem, ssem, isem):
    c = lax.axis_index("c")
    s = lax.axis_index("s")

    def work(table_hbm, out_hbm):
        r0 = s * RPT
        init_cp = pltpu.async_copy(
            table_hbm.at[pl.ds(r0, RPT)], acc_sh.at[pl.ds(r0, RPT)], isem)

        init_cp.wait()

        @pl.when(s == NS - 1)
        def _init_tail():
            pltpu.sync_copy(table_hbm.at[pl.ds(TAIL0, TAIL)],
                            acc_sh.at[pl.ds(TAIL0, TAIL)])

        plsc.subcore_barrier()

        BS = B // NSPL

        def gather_start(j, x):
            for k in range(NSPL):
                pltpu.async_copy(
                    table_hbm.at[src_v.at[pl.ds(j * B + k * BS, BS)]],
                    rows[x].at[pl.ds(k * BS, BS)], gsem.at[x * NSPL + k])

        def gather_wait(j, x):
            for k in range(NSPL):
                pltpu.make_async_copy(
                    table_hbm.at[src_v.at[pl.ds(j * B + k * BS, BS)]],
                    rows[x].at[pl.ds(k * BS, BS)],
                    gsem.at[x * NSPL + k]).wait()

        def scatter_start(j, x):
            pltpu.async_copy(rows[x], acc_sh.at[dst_v.at[j]], ssem.at[x],
                             add=True)

        def scatter_wait(x):
            pltpu.make_async_copy(
                rows[x], acc_sh.at[dst_v.at[0]], ssem.at[x]).wait()

        for half in range(NSTG):
            e0 = pl.multiple_of(s * NSTG * ESTG + half * ESTG, 128)
            c0 = pl.multiple_of(s * NSTG * NCH + half * NCH, 8)
            pltpu.sync_copy(src_hbm.at[pl.ds(e0, ESTG)], src_v)
            pltpu.sync_copy(dst_hbm.at[pl.ds(c0, NCH)], dst_v)
            pltpu.sync_copy(ew_hbm.at[pl.ds(e0, ESTG)], ew_v)

            gather_start(0, 0)

            @pl.loop(0, NCH, step=NBUF)
            def _chunks(j0):
                for x in range(NBUF):
                    j = j0 + x
                    y = (x + 1) % NBUF

                    gather_wait(j, x)

                    @pl.when(j >= NBUF - 1)
                    def _():
                        scatter_wait(y)

                    @pl.when(j + 1 < NCH)
                    def _():
                        gather_start(j + 1, y)

                    rows_x = rows[x]

                    @pl.loop(0, B, step=4)
                    def _scale(b0):
                        for d in range(4):
                            b = b0 + d
                            w = plsc.load_gather(
                                ew_v, [jnp.zeros((16,), jnp.int32) + j * B + b])
                            for k in range(H // 16):
                                sl = pl.ds(k * 16, 16)
                                rows_x[b, sl] = rows_x[b, sl] * w

                    scatter_start(j, x)

            for x in range(NBUF - 1):
                scatter_wait((NCH - 1 - x) % NBUF)

        plsc.subcore_barrier()
        pltpu.sync_copy(acc_sh.at[pl.ds(r0, RPT)], out_hbm.at[pl.ds(r0, RPT)])

        @pl.when(s == NS - 1)
        def _drain_tail():
            pltpu.sync_copy(acc_sh.at[pl.ds(TAIL0, TAIL)],
                            out_hbm.at[pl.ds(TAIL0, TAIL)])

    @pl.when(c == 0)
    def _():
        work(h2a_hbm, outa_hbm)

    @pl.when(c == 1)
    def _():
        work(h2b_hbm, outb_hbm)


R4 = 1000


def _ln_body(node_ref, agga_ref, aggb_ref, dinv_ref, b_ref, g_ref, bt_ref, out_ref):
    agg = jnp.concatenate([agga_ref[...], aggb_ref[...]], axis=1)
    y = node_ref[...] + dinv_ref[...] * agg + b_ref[...]
    mean = jnp.mean(y, axis=1, keepdims=True)
    yc = y - mean
    var = jnp.mean(yc * yc, axis=1, keepdims=True)
    out = yc * lax.rsqrt(var + 1e-5) * g_ref[...] + bt_ref[...]
    out_ref[...] = jnp.maximum(out, 0.0)


def _ln_epilogue(node, agga, aggb, dinv, b, gamma, beta):
    vec = lambda: pl.BlockSpec((1, D), lambda i: (0, 0))
    return pl.pallas_call(
        _ln_body,
        grid=(N // R4,),
        in_specs=[
            pl.BlockSpec((R4, D), lambda i: (i, 0)),
            pl.BlockSpec((R4, H), lambda i: (i, 0)),
            pl.BlockSpec((R4, H), lambda i: (i, 0)),
            pl.BlockSpec((R4, 1), lambda i: (i, 0)),
            vec(), vec(), vec(),
        ],
        out_specs=pl.BlockSpec((R4, D), lambda i: (i, 0)),
        out_shape=jax.ShapeDtypeStruct((N, D), jnp.float32),
    )(node, agga, aggb, dinv, b.reshape(1, D), gamma.reshape(1, D),
      beta.reshape(1, D))


def kernel(node, edge_index, edge_attr, batch_ptr, W, b, ln_gamma, ln_beta):
    del batch_ptr
    pad = E_PAD - E
    src = jnp.concatenate([edge_index[0], jnp.zeros((pad,), jnp.int32)])
    dst = jnp.concatenate([edge_index[1], jnp.zeros((pad,), jnp.int32)])
    ew = jnp.concatenate([edge_attr, jnp.zeros((pad,), jnp.float32)])

    deg_parts = _deg_kernel(dst, ew).reshape(NC * NS, N).T
    h2a, h2b, dinv = _matmul_scale(node, W, deg_parts)
    agga, aggb = _agg_kernel(h2a, h2b, src, dst.reshape(E_PAD // B, B), ew)
    return _ln_epilogue(node, agga, aggb, dinv, b, ln_gamma, ln_beta)


# speedup vs baseline: 1.0023x; 1.0023x over previous
import dataclasses
import functools

import jax
import jax.numpy as jnp
from jax import lax
from jax.experimental import pallas as pl
from jax.experimental.pallas import tpu as pltpu
from jax.experimental.pallas import tpu_sc as plsc

N = 10000
E = 160000
D = 256
H = 128

NC = 2
NS = 16
E_PAD = 163840
B = 128

_mesh = plsc.VectorSubcoreMesh(
    core_axis_name="c", subcore_axis_name="s", num_cores=NC, num_subcores=NS
)

_sc_params = pltpu.CompilerParams()
if "needs_layout_passes" in pltpu.CompilerParams.__dataclass_fields__:
    _sc_params = dataclasses.replace(_sc_params, needs_layout_passes=False)


EPT1 = E_PAD // (NC * NS)


@functools.partial(
    pl.kernel,
    out_type=jax.ShapeDtypeStruct((NC * NS * N,), jnp.float32),
    mesh=_mesh,
    scratch_types=[
        pltpu.VMEM((N,), jnp.float32),
        pltpu.VMEM((EPT1,), jnp.int32),
        pltpu.VMEM((EPT1,), jnp.float32),
    ],
    compiler_params=_sc_params,
)
def _deg_kernel(dst_hbm, ew_hbm, out_hbm, part_v, dst_v, ew_v):
    wid = lax.axis_index("s") * NC + lax.axis_index("c")
    base = wid * EPT1

    @pl.loop(0, N, step=16)
    def _zero(i):
        part_v[pl.ds(i, 16)] = jnp.zeros((16,), jnp.float32)

    pltpu.sync_copy(dst_hbm.at[pl.ds(base, EPT1)], dst_v)
    pltpu.sync_copy(ew_hbm.at[pl.ds(base, EPT1)], ew_v)

    @pl.loop(0, EPT1, step=16)
    def _acc(k):
        plsc.addupdate_scatter(part_v, [dst_v[pl.ds(k, 16)]], ew_v[pl.ds(k, 16)])

    pltpu.sync_copy(part_v, out_hbm.at[pl.ds(wid * N, N)])


R2 = 1000


def _mm_body(node_ref, w_ref, degp_ref, h2a_ref, h2b_ref, dinv_ref):
    deg = jnp.sum(degp_ref[...], axis=1) + 1.0
    dinv = jnp.where(deg > 0, lax.rsqrt(deg), 0.0)
    h = jnp.dot(node_ref[...].astype(jnp.bfloat16),
                w_ref[...].astype(jnp.bfloat16),
                preferred_element_type=jnp.float32)
    h2 = h * dinv[:, None]
    h2a_ref[...] = h2[:, :H]
    h2b_ref[...] = h2[:, H:]
    dinv_ref[...] = dinv[:, None]


def _matmul_scale(node, w, deg_parts):
    return pl.pallas_call(
        _mm_body,
        grid=(N // R2,),
        in_specs=[
            pl.BlockSpec((R2, D), lambda i: (i, 0)),
            pl.BlockSpec((D, D), lambda i: (0, 0)),
            pl.BlockSpec((R2, NC * NS), lambda i: (i, 0)),
        ],
        out_specs=[
            pl.BlockSpec((R2, H), lambda i: (i, 0)),
            pl.BlockSpec((R2, H), lambda i: (i, 0)),
            pl.BlockSpec((R2, 1), lambda i: (i, 0)),
        ],
        out_shape=[
            jax.ShapeDtypeStruct((N, H), jnp.float32),
            jax.ShapeDtypeStruct((N, H), jnp.float32),
            jax.ShapeDtypeStruct((N, 1), jnp.float32),
        ],
    )(node, w, deg_parts)


EPT3 = E_PAD // NS
NSTG = 2
NCH = EPT3 // (B * NSTG)
NBUF = 2
ESTG = NCH * B
RPT = 624
TAIL0 = NS * RPT
TAIL = N - TAIL0


@functools.partial(
    pl.kernel,
    out_type=(
        jax.ShapeDtypeStruct((N, H), jnp.float32),
        jax.ShapeDtypeStruct((N, H), jnp.float32),
    ),
    mesh=_mesh,
    scratch_types=[
        pltpu.VMEM_SHARED((N, H), jnp.float32),
        pltpu.VMEM((ESTG,), jnp.int32),
        pltpu.VMEM((NCH, B), jnp.int32),
        pltpu.VMEM((ESTG,), jnp.float32),
        [pltpu.VMEM((B, H), jnp.float32) for _ in range(NBUF)],
        pltpu.SemaphoreType.DMA((NBUF,)),
        pltpu.SemaphoreType.DMA((NBUF,)),
        pltpu.SemaphoreType.DMA,
    ],
    compiler_params=_sc_params,
)
def _agg_kernel(h2a_hbm, h2b_hbm, src_hbm, dst_hbm, ew_hbm,
                outa_hbm, outb_hbm, acc_sh, src_v, dst_v, ew_v, rows,
                gsem, ssem, isem):
    c = lax.axis_index("c")
    s = lax.axis_index("s")

    def work(table_hbm, out_hbm):
        r0 = s * RPT
        pltpu.async_copy(
            table_hbm.at[pl.ds(r0, RPT)], acc_sh.at[pl.ds(r0, RPT)],
            isem).wait()

        @pl.when(s == NS - 1)
        def _init_tail():
            pltpu.sync_copy(table_hbm.at[pl.ds(TAIL0, TAIL)],
                            acc_sh.at[pl.ds(TAIL0, TAIL)])

        plsc.subcore_barrier()

        def gather_start(j, x):
            pltpu.async_copy(table_hbm.at[src_v.at[pl.ds(j * B, B)]],
                             rows[x], gsem.at[x])

        def gather_wait(j, x):
            pltpu.make_async_copy(table_hbm.at[src_v.at[pl.ds(j * B, B)]],
                                  rows[x], gsem.at[x]).wait()

        def scatter_start(j, x):
            pltpu.async_copy(rows[x], acc_sh.at[dst_v.at[j]], ssem.at[x],
                             add=True)

        def scatter_wait(x):
            pltpu.make_async_copy(
                rows[x], acc_sh.at[dst_v.at[0]], ssem.at[x]).wait()

        for half in range(NSTG):
            e0 = pl.multiple_of(s * NSTG * ESTG + half * ESTG, 128)
            c0 = pl.multiple_of(s * NSTG * NCH + half * NCH, 8)
            pltpu.sync_copy(src_hbm.at[pl.ds(e0, ESTG)], src_v)
            pltpu.sync_copy(dst_hbm.at[pl.ds(c0, NCH)], dst_v)
            pltpu.sync_copy(ew_hbm.at[pl.ds(e0, ESTG)], ew_v)

            gather_start(0, 0)

            @pl.loop(0, NCH, step=NBUF)
            def _chunks(j0):
                for x in range(NBUF):
                    j = j0 + x
                    y = (x + 1) % NBUF

                    gather_wait(j, x)

                    @pl.when(j >= NBUF - 1)
                    def _():
                        scatter_wait(y)

                    @pl.when(j + 1 < NCH)
                    def _():
                        gather_start(j + 1, y)

                    rows_x = rows[x]

                    @pl.loop(0, B, step=4)
                    def _scale(b0):
                        for d in range(4):
                            b = b0 + d
                            w = plsc.load_gather(
                                ew_v, [jnp.zeros((16,), jnp.int32) + j * B + b])
                            for k in range(H // 16):
                                sl = pl.ds(k * 16, 16)
                                rows_x[b, sl] = rows_x[b, sl] * w

                    scatter_start(j, x)

            for x in range(NBUF - 1):
                scatter_wait((NCH - 1 - x) % NBUF)

        plsc.subcore_barrier()
        pltpu.sync_copy(acc_sh.at[pl.ds(r0, RPT)], out_hbm.at[pl.ds(r0, RPT)])

        @pl.when(s == NS - 1)
        def _drain_tail():
            pltpu.sync_copy(acc_sh.at[pl.ds(TAIL0, TAIL)],
                            out_hbm.at[pl.ds(TAIL0, TAIL)])

    @pl.when(c == 0)
    def _():
        work(h2a_hbm, outa_hbm)

    @pl.when(c == 1)
    def _():
        work(h2b_hbm, outb_hbm)


R4 = 1000


def _ln_body(node_ref, agga_ref, aggb_ref, dinv_ref, b_ref, g_ref, bt_ref,
             out_ref):
    agg = jnp.concatenate([agga_ref[...], aggb_ref[...]], axis=1)
    y = node_ref[...] + dinv_ref[...] * agg + b_ref[...]
    mean = jnp.mean(y, axis=1, keepdims=True)
    yc = y - mean
    var = jnp.mean(yc * yc, axis=1, keepdims=True)
    out = yc * lax.rsqrt(var + 1e-5) * g_ref[...] + bt_ref[...]
    out_ref[...] = jnp.maximum(out, 0.0)


def _ln_epilogue(node, agga, aggb, dinv, b, gamma, beta):
    vec = lambda: pl.BlockSpec((1, D), lambda i: (0, 0))
    return pl.pallas_call(
        _ln_body,
        grid=(N // R4,),
        in_specs=[
            pl.BlockSpec((R4, D), lambda i: (i, 0)),
            pl.BlockSpec((R4, H), lambda i: (i, 0)),
            pl.BlockSpec((R4, H), lambda i: (i, 0)),
            pl.BlockSpec((R4, 1), lambda i: (i, 0)),
            vec(), vec(), vec(),
        ],
        out_specs=pl.BlockSpec((R4, D), lambda i: (i, 0)),
        out_shape=jax.ShapeDtypeStruct((N, D), jnp.float32),
    )(node, agga, aggb, dinv, b.reshape(1, D), gamma.reshape(1, D),
      beta.reshape(1, D))


def kernel(node, edge_index, edge_attr, batch_ptr, W, b, ln_gamma, ln_beta):
    del batch_ptr
    pad = E_PAD - E
    src = jnp.concatenate([edge_index[0], jnp.zeros((pad,), jnp.int32)])
    dst = jnp.concatenate([edge_index[1], jnp.zeros((pad,), jnp.int32)])
    ew = jnp.concatenate([edge_attr, jnp.zeros((pad,), jnp.float32)])

    deg_parts = _deg_kernel(dst, ew).reshape(NC * NS, N).T
    h2a, h2b, dinv = _matmul_scale(node, W, deg_parts)
    agga, aggb = _agg_kernel(h2a, h2b, src, dst.reshape(E_PAD // B, B), ew)
    return _ln_epilogue(node, agga, aggb, dinv, b, ln_gamma, ln_beta)


# speedup vs baseline: 1.0526x; 1.0501x over previous
import dataclasses
import functools

import jax
import jax.numpy as jnp
from jax import lax
from jax.experimental import pallas as pl
from jax.experimental.pallas import tpu as pltpu
from jax.experimental.pallas import tpu_sc as plsc

N = 10000
E = 160000
D = 256
H = 128

NC = 2
NS = 16
E_PAD = 163840
B = 128

_mesh = plsc.VectorSubcoreMesh(
    core_axis_name="c", subcore_axis_name="s", num_cores=NC, num_subcores=NS
)

_sc_params = pltpu.CompilerParams()
if "needs_layout_passes" in pltpu.CompilerParams.__dataclass_fields__:
    _sc_params = dataclasses.replace(_sc_params, needs_layout_passes=False)


EPT1 = E_PAD // (NC * NS)


@functools.partial(
    pl.kernel,
    out_type=jax.ShapeDtypeStruct((NC * NS * N,), jnp.float32),
    mesh=_mesh,
    scratch_types=[
        pltpu.VMEM((N,), jnp.float32),
        pltpu.VMEM((EPT1,), jnp.int32),
        pltpu.VMEM((EPT1,), jnp.float32),
    ],
    compiler_params=_sc_params,
)
def _deg_kernel(dst_hbm, ew_hbm, out_hbm, part_v, dst_v, ew_v):
    wid = lax.axis_index("s") * NC + lax.axis_index("c")
    base = wid * EPT1

    @pl.loop(0, N, step=16)
    def _zero(i):
        part_v[pl.ds(i, 16)] = jnp.zeros((16,), jnp.float32)

    pltpu.sync_copy(dst_hbm.at[pl.ds(base, EPT1)], dst_v)
    pltpu.sync_copy(ew_hbm.at[pl.ds(base, EPT1)], ew_v)

    @pl.loop(0, EPT1, step=16)
    def _acc(k):
        plsc.addupdate_scatter(part_v, [dst_v[pl.ds(k, 16)]], ew_v[pl.ds(k, 16)])

    pltpu.sync_copy(part_v, out_hbm.at[pl.ds(wid * N, N)])


R2 = 1000


def _mm_body(node_ref, w_ref, degp_ref, h2a_ref, h2b_ref, dinv_ref):
    deg = jnp.sum(degp_ref[...], axis=1) + 1.0
    dinv = jnp.where(deg > 0, lax.rsqrt(deg), 0.0)
    h = jnp.dot(node_ref[...], w_ref[...], preferred_element_type=jnp.float32)
    h2 = h * dinv[:, None]
    h2a_ref[...] = h2[:, :H]
    h2b_ref[...] = h2[:, H:]
    dinv_ref[...] = dinv[:, None]


def _matmul_scale(node, w, deg_parts):
    return pl.pallas_call(
        _mm_body,
        grid=(N // R2,),
        in_specs=[
            pl.BlockSpec((R2, D), lambda i: (i, 0)),
            pl.BlockSpec((D, D), lambda i: (0, 0)),
            pl.BlockSpec((R2, NC * NS), lambda i: (i, 0)),
        ],
        out_specs=[
            pl.BlockSpec((R2, H), lambda i: (i, 0)),
            pl.BlockSpec((R2, H), lambda i: (i, 0)),
            pl.BlockSpec((R2, 1), lambda i: (i, 0)),
        ],
        out_shape=[
            jax.ShapeDtypeStruct((N, H), jnp.float32),
            jax.ShapeDtypeStruct((N, H), jnp.float32),
            jax.ShapeDtypeStruct((N, 1), jnp.float32),
        ],
    )(node, w, deg_parts)


EPT3 = E_PAD // NS
NSTG = 2
NCH = EPT3 // (B * NSTG)
NBUF = 2
ESTG = NCH * B
RPT = 624
TAIL0 = NS * RPT
TAIL = N - TAIL0


@functools.partial(
    pl.kernel,
    out_type=(
        jax.ShapeDtypeStruct((N, H), jnp.float32),
        jax.ShapeDtypeStruct((N, H), jnp.float32),
    ),
    mesh=_mesh,
    scratch_types=[
        pltpu.VMEM_SHARED((N, H), jnp.float32),
        pltpu.VMEM((ESTG,), jnp.int32),
        pltpu.VMEM((NCH, B), jnp.int32),
        pltpu.VMEM((ESTG,), jnp.float32),
        [pltpu.VMEM((B, H), jnp.float32) for _ in range(NBUF)],
        pltpu.SemaphoreType.DMA((NBUF,)),
        pltpu.SemaphoreType.DMA((NBUF,)),
        pltpu.SemaphoreType.DMA,
    ],
    compiler_params=_sc_params,
)
def _agg_kernel(h2a_hbm, h2b_hbm, src_hbm, dst_hbm, ew_hbm,
                outa_hbm, outb_hbm, acc_sh, src_v, dst_v, ew_v, rows,
                gsem, ssem, isem):
    c = lax.axis_index("c")
    s = lax.axis_index("s")

    def work(table_hbm, out_hbm):
        r0 = s * RPT
        pltpu.async_copy(
            table_hbm.at[pl.ds(r0, RPT)], acc_sh.at[pl.ds(r0, RPT)],
            isem).wait()

        @pl.when(s == NS - 1)
        def _init_tail():
            pltpu.sync_copy(table_hbm.at[pl.ds(TAIL0, TAIL)],
                            acc_sh.at[pl.ds(TAIL0, TAIL)])

        plsc.subcore_barrier()

        def gather_start(j, x):
            pltpu.async_copy(table_hbm.at[src_v.at[pl.ds(j * B, B)]],
                             rows[x], gsem.at[x])

        def gather_wait(j, x):
            pltpu.make_async_copy(table_hbm.at[src_v.at[pl.ds(j * B, B)]],
                                  rows[x], gsem.at[x]).wait()

        def scatter_start(j, x):
            pltpu.async_copy(rows[x], acc_sh.at[dst_v.at[j]], ssem.at[x],
                             add=True)

        def scatter_wait(x):
            pltpu.make_async_copy(
                rows[x], acc_sh.at[dst_v.at[0]], ssem.at[x]).wait()

        for half in range(NSTG):
            e0 = pl.multiple_of(s * NSTG * ESTG + half * ESTG, 128)
            c0 = pl.multiple_of(s * NSTG * NCH + half * NCH, 8)
            pltpu.sync_copy(src_hbm.at[pl.ds(e0, ESTG)], src_v)
            pltpu.sync_copy(dst_hbm.at[pl.ds(c0, NCH)], dst_v)
            pltpu.sync_copy(ew_hbm.at[pl.ds(e0, ESTG)], ew_v)

            gather_start(0, 0)

            @pl.loop(0, NCH, step=NBUF)
            def _chunks(j0):
                for x in range(NBUF):
                    j = j0 + x
                    y = (x + 1) % NBUF

                    gather_wait(j, x)

                    @pl.when(j >= NBUF - 1)
                    def _():
                        scatter_wait(y)

                    @pl.when(j + 1 < NCH)
                    def _():
                        gather_start(j + 1, y)

                    rows_x = rows[x]

                    @plsc.parallel_loop(0, B, step=1, unroll=4)
                    def _scale(b):
                        w = plsc.load_gather(
                            ew_v, [jnp.zeros((16,), jnp.int32) + j * B + b])
                        for k in range(H // 16):
                            sl = pl.ds(k * 16, 16)
                            rows_x[b, sl] = rows_x[b, sl] * w

                    scatter_start(j, x)

            for x in range(NBUF - 1):
                scatter_wait((NCH - 1 - x) % NBUF)

        plsc.subcore_barrier()
        pltpu.sync_copy(acc_sh.at[pl.ds(r0, RPT)], out_hbm.at[pl.ds(r0, RPT)])

        @pl.when(s == NS - 1)
        def _drain_tail():
            pltpu.sync_copy(acc_sh.at[pl.ds(TAIL0, TAIL)],
                            out_hbm.at[pl.ds(TAIL0, TAIL)])

    @pl.when(c == 0)
    def _():
        work(h2a_hbm, outa_hbm)

    @pl.when(c == 1)
    def _():
        work(h2b_hbm, outb_hbm)


R4 = 1000


def _ln_body(node_ref, agga_ref, aggb_ref, dinv_ref, b_ref, g_ref, bt_ref,
             out_ref):
    agg = jnp.concatenate([agga_ref[...], aggb_ref[...]], axis=1)
    y = node_ref[...] + dinv_ref[...] * agg + b_ref[...]
    mean = jnp.mean(y, axis=1, keepdims=True)
    yc = y - mean
    var = jnp.mean(yc * yc, axis=1, keepdims=True)
    out = yc * lax.rsqrt(var + 1e-5) * g_ref[...] + bt_ref[...]
    out_ref[...] = jnp.maximum(out, 0.0)


def _ln_epilogue(node, agga, aggb, dinv, b, gamma, beta):
    vec = lambda: pl.BlockSpec((1, D), lambda i: (0, 0))
    return pl.pallas_call(
        _ln_body,
        grid=(N // R4,),
        in_specs=[
            pl.BlockSpec((R4, D), lambda i: (i, 0)),
            pl.BlockSpec((R4, H), lambda i: (i, 0)),
            pl.BlockSpec((R4, H), lambda i: (i, 0)),
            pl.BlockSpec((R4, 1), lambda i: (i, 0)),
            vec(), vec(), vec(),
        ],
        out_specs=pl.BlockSpec((R4, D), lambda i: (i, 0)),
        out_shape=jax.ShapeDtypeStruct((N, D), jnp.float32),
    )(node, agga, aggb, dinv, b.reshape(1, D), gamma.reshape(1, D),
      beta.reshape(1, D))


def kernel(node, edge_index, edge_attr, batch_ptr, W, b, ln_gamma, ln_beta):
    del batch_ptr
    pad = E_PAD - E
    src = jnp.concatenate([edge_index[0], jnp.zeros((pad,), jnp.int32)])
    dst = jnp.concatenate([edge_index[1], jnp.zeros((pad,), jnp.int32)])
    ew = jnp.concatenate([edge_attr, jnp.zeros((pad,), jnp.float32)])

    deg_parts = _deg_kernel(dst, ew).reshape(NC * NS, N).T
    h2a, h2b, dinv = _matmul_scale(node, W, deg_parts)
    agga, aggb = _agg_kernel(h2a, h2b, src, dst.reshape(E_PAD // B, B), ew)
    return _ln_epilogue(node, agga, aggb, dinv, b, ln_gamma, ln_beta)


# speedup vs baseline: 1.0829x; 1.0288x over previous
import dataclasses
import functools

import jax
import jax.numpy as jnp
from jax import lax
from jax.experimental import pallas as pl
from jax.experimental.pallas import tpu as pltpu
from jax.experimental.pallas import tpu_sc as plsc

N = 10000
E = 160000
D = 256
H = 128

NC = 2
NS = 16
E_PAD = 163840
B = 128

_mesh = plsc.VectorSubcoreMesh(
    core_axis_name="c", subcore_axis_name="s", num_cores=NC, num_subcores=NS
)

_sc_params = pltpu.CompilerParams()
if "needs_layout_passes" in pltpu.CompilerParams.__dataclass_fields__:
    _sc_params = dataclasses.replace(_sc_params, needs_layout_passes=False)


EPT1 = E_PAD // (NC * NS)


@functools.partial(
    pl.kernel,
    out_type=jax.ShapeDtypeStruct((NC * NS * N,), jnp.float32),
    mesh=_mesh,
    scratch_types=[
        pltpu.VMEM((N,), jnp.float32),
        pltpu.VMEM((EPT1,), jnp.int32),
        pltpu.VMEM((EPT1,), jnp.float32),
    ],
    compiler_params=_sc_params,
)
def _deg_kernel(dst_hbm, ew_hbm, out_hbm, part_v, dst_v, ew_v):
    wid = lax.axis_index("s") * NC + lax.axis_index("c")
    base = wid * EPT1

    @pl.loop(0, N, step=16)
    def _zero(i):
        part_v[pl.ds(i, 16)] = jnp.zeros((16,), jnp.float32)

    pltpu.sync_copy(dst_hbm.at[pl.ds(base, EPT1)], dst_v)
    pltpu.sync_copy(ew_hbm.at[pl.ds(base, EPT1)], ew_v)

    @pl.loop(0, EPT1, step=16)
    def _acc(k):
        plsc.addupdate_scatter(part_v, [dst_v[pl.ds(k, 16)]], ew_v[pl.ds(k, 16)])

    pltpu.sync_copy(part_v, out_hbm.at[pl.ds(wid * N, N)])


R2 = 1000


def _mm_body(node_ref, w_ref, degp_ref, h2a_ref, h2b_ref, dinv_ref):
    deg = jnp.sum(degp_ref[...], axis=1) + 1.0
    dinv = jnp.where(deg > 0, lax.rsqrt(deg), 0.0)
    h = jnp.dot(node_ref[...], w_ref[...], preferred_element_type=jnp.float32)
    h2 = h * dinv[:, None]
    h2a_ref[...] = h2[:, :H]
    h2b_ref[...] = h2[:, H:]
    dinv_ref[...] = dinv[:, None]


def _matmul_scale(node, w, deg_parts):
    return pl.pallas_call(
        _mm_body,
        grid=(N // R2,),
        in_specs=[
            pl.BlockSpec((R2, D), lambda i: (i, 0)),
            pl.BlockSpec((D, D), lambda i: (0, 0)),
            pl.BlockSpec((R2, NC * NS), lambda i: (i, 0)),
        ],
        out_specs=[
            pl.BlockSpec((R2, H), lambda i: (i, 0)),
            pl.BlockSpec((R2, H), lambda i: (i, 0)),
            pl.BlockSpec((R2, 1), lambda i: (i, 0)),
        ],
        out_shape=[
            jax.ShapeDtypeStruct((N, H), jnp.float32),
            jax.ShapeDtypeStruct((N, H), jnp.float32),
            jax.ShapeDtypeStruct((N, 1), jnp.float32),
        ],
    )(node, w, deg_parts)


EPT3 = E_PAD // NS
NSTG = 2
NCH = EPT3 // (B * NSTG)
NBUF = 2
ESTG = NCH * B
RPT = 624
TAIL0 = NS * RPT
TAIL = N - TAIL0


@functools.partial(
    pl.kernel,
    out_type=(
        jax.ShapeDtypeStruct((N, H), jnp.float32),
        jax.ShapeDtypeStruct((N, H), jnp.float32),
    ),
    mesh=_mesh,
    scratch_types=[
        pltpu.VMEM_SHARED((N, H), jnp.float32),
        pltpu.VMEM((ESTG,), jnp.int32),
        pltpu.VMEM((NCH, B), jnp.int32),
        pltpu.VMEM((ESTG,), jnp.float32),
        [pltpu.VMEM((B, H), jnp.float32) for _ in range(NBUF)],
        pltpu.SemaphoreType.DMA((NBUF,)),
        pltpu.SemaphoreType.DMA((NBUF,)),
        pltpu.SemaphoreType.DMA,
    ],
    compiler_params=_sc_params,
)
def _agg_kernel(h2a_hbm, h2b_hbm, src_hbm, dst_hbm, ew_hbm,
                outa_hbm, outb_hbm, acc_sh, src_v, dst_v, ew_v, rows,
                gsem, ssem, isem):
    c = lax.axis_index("c")
    s = lax.axis_index("s")

    def work(table_hbm, out_hbm):
        r0 = s * RPT
        pltpu.async_copy(
            table_hbm.at[pl.ds(r0, RPT)], acc_sh.at[pl.ds(r0, RPT)],
            isem).wait()

        @pl.when(s == NS - 1)
        def _init_tail():
            pltpu.sync_copy(table_hbm.at[pl.ds(TAIL0, TAIL)],
                            acc_sh.at[pl.ds(TAIL0, TAIL)])

        plsc.subcore_barrier()

        def gather_start(j, x):
            pltpu.async_copy(table_hbm.at[src_v.at[pl.ds(j * B, B)]],
                             rows[x], gsem.at[x])

        def gather_wait(j, x):
            pltpu.make_async_copy(table_hbm.at[src_v.at[pl.ds(j * B, B)]],
                                  rows[x], gsem.at[x]).wait()

        def scatter_start(j, x):
            pltpu.async_copy(rows[x], acc_sh.at[dst_v.at[j]], ssem.at[x],
                             add=True)

        def scatter_wait(x):
            pltpu.make_async_copy(
                rows[x], acc_sh.at[dst_v.at[0]], ssem.at[x]).wait()

        for half in range(NSTG):
            e0 = pl.multiple_of(s * NSTG * ESTG + half * ESTG, 128)
            c0 = pl.multiple_of(s * NSTG * NCH + half * NCH, 8)
            pltpu.sync_copy(src_hbm.at[pl.ds(e0, ESTG)], src_v)
            pltpu.sync_copy(dst_hbm.at[pl.ds(c0, NCH)], dst_v)
            pltpu.sync_copy(ew_hbm.at[pl.ds(e0, ESTG)], ew_v)

            gather_start(0, 0)

            @pl.loop(0, NCH, step=NBUF)
            def _chunks(j0):
                for x in range(NBUF):
                    j = j0 + x
                    y = (x + 1) % NBUF

                    @pl.when(j >= NBUF - 1)
                    def _():
                        scatter_wait(y)

                    @pl.when(j + 1 < NCH)
                    def _():
                        gather_start(j + 1, y)

                    gather_wait(j, x)

                    rows_x = rows[x]

                    @plsc.parallel_loop(0, B, step=1, unroll=4)
                    def _scale(b):
                        w = plsc.load_gather(
                            ew_v, [jnp.zeros((16,), jnp.int32) + j * B + b])
                        for k in range(H // 16):
                            sl = pl.ds(k * 16, 16)
                            rows_x[b, sl] = rows_x[b, sl] * w

                    scatter_start(j, x)

            for x in range(NBUF - 1):
                scatter_wait((NCH - 1 - x) % NBUF)

        plsc.subcore_barrier()
        pltpu.sync_copy(acc_sh.at[pl.ds(r0, RPT)], out_hbm.at[pl.ds(r0, RPT)])

        @pl.when(s == NS - 1)
        def _drain_tail():
            pltpu.sync_copy(acc_sh.at[pl.ds(TAIL0, TAIL)],
                            out_hbm.at[pl.ds(TAIL0, TAIL)])

    @pl.when(c == 0)
    def _():
        work(h2a_hbm, outa_hbm)

    @pl.when(c == 1)
    def _():
        work(h2b_hbm, outb_hbm)


R4 = 1000


def _ln_body(node_ref, agga_ref, aggb_ref, dinv_ref, b_ref, g_ref, bt_ref,
             out_ref):
    agg = jnp.concatenate([agga_ref[...], aggb_ref[...]], axis=1)
    y = node_ref[...] + dinv_ref[...] * agg + b_ref[...]
    mean = jnp.mean(y, axis=1, keepdims=True)
    yc = y - mean
    var = jnp.mean(yc * yc, axis=1, keepdims=True)
    out = yc * lax.rsqrt(var + 1e-5) * g_ref[...] + bt_ref[...]
    out_ref[...] = jnp.maximum(out, 0.0)


def _ln_epilogue(node, agga, aggb, dinv, b, gamma, beta):
    vec = lambda: pl.BlockSpec((1, D), lambda i: (0, 0))
    return pl.pallas_call(
        _ln_body,
        grid=(N // R4,),
        in_specs=[
            pl.BlockSpec((R4, D), lambda i: (i, 0)),
            pl.BlockSpec((R4, H), lambda i: (i, 0)),
            pl.BlockSpec((R4, H), lambda i: (i, 0)),
            pl.BlockSpec((R4, 1), lambda i: (i, 0)),
            vec(), vec(), vec(),
        ],
        out_specs=pl.BlockSpec((R4, D), lambda i: (i, 0)),
        out_shape=jax.ShapeDtypeStruct((N, D), jnp.float32),
    )(node, agga, aggb, dinv, b.reshape(1, D), gamma.reshape(1, D),
      beta.reshape(1, D))


def kernel(node, edge_index, edge_attr, batch_ptr, W, b, ln_gamma, ln_beta):
    del batch_ptr
    pad = E_PAD - E
    src = jnp.concatenate([edge_index[0], jnp.zeros((pad,), jnp.int32)])
    dst = jnp.concatenate([edge_index[1], jnp.zeros((pad,), jnp.int32)])
    ew = jnp.concatenate([edge_attr, jnp.zeros((pad,), jnp.float32)])

    deg_parts = _deg_kernel(dst, ew).reshape(NC * NS, N).T
    h2a, h2b, dinv = _matmul_scale(node, W, deg_parts)
    agga, aggb = _agg_kernel(h2a, h2b, src, dst.reshape(E_PAD // B, B), ew)
    return _ln_epilogue(node, agga, aggb, dinv, b, ln_gamma, ln_beta)
